# Initial kernel scaffold; baseline (speedup 1.0000x reference)
import jax
import jax.numpy as jnp
from jax.experimental import pallas as pl


def kernel(x, w_list, mask, x0, y0, x1, y1):
    raise NotImplementedError("write your pallas kernel here")


# speedup vs baseline: 75.2634x; 75.2634x over previous
import functools

import jax
import jax.numpy as jnp
from jax import lax
from jax.experimental import pallas as pl
from jax.experimental.pallas import tpu as pltpu
from jax.experimental.pallas import tpu_sc as plsc

P = 18
PH = 224
PHP = 225
H, W = 256, 512
NPIX = H * W
CH = 64
NCHUNK = NPIX // CH
NC, NS = 2, 16
NW = NC * NS
CPW = NCHUNK // NW
BM_W = 40
THR = 1e-5
EPS = 1e-12


def _sc_kernel(table, wf, x0f, y0f, bitmap, out,
               G, idxb, x0v, y0v, wv, bmv, acc, den, sem):
    wid = lax.axis_index("s") * NC + lax.axis_index("c")
    lane = lax.iota(jnp.int32, 16)
    zf = jnp.zeros((16,), jnp.float32)

    def chunk_body(i, _):
        chunk = wid + i * NW
        base = chunk * CH
        for c in range(8):
            for v in range(4):
                acc[c, pl.ds(16 * v, 16)] = zf
        for v in range(4):
            den[pl.ds(16 * v, 16)] = zf
        pltpu.sync_copy(bitmap.at[pl.ds(chunk * BM_W, BM_W)], bmv)

        def patch_body(p, _):
            bit = bmv[pl.ds(p, 16)][0]

            @pl.when(bit != 0)
            def _():
                pltpu.sync_copy(x0f.at[pl.ds(p * NPIX + base, CH)], x0v)
                pltpu.sync_copy(y0f.at[pl.ds(p * NPIX + base, CH)], y0v)
                pltpu.sync_copy(wf.at[pl.ds((p * NPIX + base) * 4, CH * 4)], wv)
                pb = p * (PH * PHP)
                for v in range(4):
                    xx0 = x0v[pl.ds(16 * v, 16)]
                    yy0 = y0v[pl.ds(16 * v, 16)]
                    xx1 = jnp.minimum(xx0 + 1, PH - 1)
                    yy1 = jnp.minimum(yy0 + 1, PH - 1)
                    c0 = pb + xx0 * PHP
                    c1 = pb + xx1 * PHP
                    idxb[0, pl.ds(16 * v, 16)] = c0 + yy0
                    idxb[1, pl.ds(16 * v, 16)] = c0 + yy1
                    idxb[2, pl.ds(16 * v, 16)] = c1 + yy0
                    idxb[3, pl.ds(16 * v, 16)] = c1 + yy1
                descs = [
                    pltpu.async_copy(table.at[idxb.at[k]],
                                     G.at[pl.ds(k * CH, CH)], sem)
                    for k in range(4)
                ]
                for d in descs:
                    d.wait()
                for v in range(4):
                    pix = lane + (16 * v)
                    tw = []
                    for k in range(4):
                        wk = plsc.load_gather(wv, [pix * 4 + k])
                        tw.append(jnp.where(wk > THR, wk, 0.0))
                    den[pl.ds(16 * v, 16)] = (den[pl.ds(16 * v, 16)]
                                              + tw[0] + tw[1] + tw[2] + tw[3])
                    for c in range(8):
                        cv = jnp.full((16,), c, jnp.int32)
                        s = acc[c, pl.ds(16 * v, 16)]
                        for k in range(4):
                            val = plsc.load_gather(G, [pix + k * CH, cv])
                            s = s + tw[k] * val
                        acc[c, pl.ds(16 * v, 16)] = s

        lax.fori_loop(0, P, patch_body, None)

        for v in range(4):
            dv = jnp.maximum(den[pl.ds(16 * v, 16)], EPS)
            for c in range(8):
                acc[c, pl.ds(16 * v, 16)] = acc[c, pl.ds(16 * v, 16)] / dv
        for c in range(8):
            pltpu.sync_copy(acc.at[c], out.at[pl.ds(c * NPIX + base, CH)])

    lax.fori_loop(0, CPW, chunk_body, None)


_pers2equi_sc = functools.partial(
    pl.kernel,
    out_type=jax.ShapeDtypeStruct((8 * NPIX,), jnp.float32),
    mesh=plsc.VectorSubcoreMesh(core_axis_name="c", subcore_axis_name="s"),
    compiler_params=pltpu.CompilerParams(needs_layout_passes=False,
                                         use_tc_tiling_on_sc=False),
    scratch_types=[
        pltpu.VMEM((4 * CH, 8), jnp.float32),
        pltpu.VMEM((4, CH), jnp.int32),
        pltpu.VMEM((CH,), jnp.int32),
        pltpu.VMEM((CH,), jnp.int32),
        pltpu.VMEM((CH * 4,), jnp.float32),
        pltpu.VMEM((BM_W,), jnp.int32),
        pltpu.VMEM((8, CH), jnp.float32),
        pltpu.VMEM((CH,), jnp.float32),
        pltpu.SemaphoreType.DMA,
    ],
)(_sc_kernel)


def kernel(x, w_list, mask, x0, y0, x1, y1):
    del x1, y1
    xt = jnp.transpose(x[0], (3, 2, 1, 0))
    xt = jnp.pad(xt, ((0, 0), (0, 0), (0, 1), (0, 0)))
    table = xt.reshape(P * PH * PHP, 8)
    wf = w_list.reshape(P * NPIX * 4)
    x0f = x0.astype(jnp.int32).reshape(P * NPIX)
    y0f = y0.astype(jnp.int32).reshape(P * NPIX)
    bm = mask.reshape(P, NCHUNK, CH).max(-1).astype(jnp.int32)
    bm = jnp.pad(bm.T, ((0, 0), (0, BM_W - P))).reshape(NCHUNK * BM_W)
    out = _pers2equi_sc(table, wf, x0f, y0f, bm)
    return out.reshape(1, 8, H, W)


# speedup vs baseline: 83.1532x; 1.1048x over previous
import functools

import jax
import jax.numpy as jnp
from jax import lax
from jax.experimental import pallas as pl
from jax.experimental.pallas import tpu as pltpu
from jax.experimental.pallas import tpu_sc as plsc

P = 18
PH = 224
PHP = 225
H, W = 256, 512
NPIX = H * W
CH = 64
NCHUNK = NPIX // CH
NC, NS = 2, 16
NW = NC * NS
CPW = NCHUNK // NW
BM_W = 40
GSLOT = 4 * CH
THR = 1e-5
EPS = 1e-12


def _sc_kernel(table, wf, x0f, y0f, bitmap, out,
               G, idxb, x0b, y0b, wb, bmall, acc, den,
               sem_aux, sem_g, sem_out):
    wid = lax.axis_index("s") * NC + lax.axis_index("c")
    lane = lax.iota(jnp.int32, 16)
    zf = jnp.zeros((16,), jnp.float32)

    pltpu.sync_copy(bitmap.at[pl.ds(wid * (CPW * BM_W), CPW * BM_W)], bmall)

    def bit_of(ci, p):
        return bmall[pl.ds(ci * BM_W + p, 16)][0]

    def fire_aux(ci):
        base = (wid + ci * NW) * CH
        qa = lax.rem(ci, 3)

        def pb_(p, _):
            @pl.when(bit_of(ci, p) != 0)
            def _():
                s = (qa * P + p) * CH
                pltpu.async_copy(x0f.at[pl.ds(p * NPIX + base, CH)],
                                 x0b.at[pl.ds(s, CH)], sem_aux)
                pltpu.async_copy(y0f.at[pl.ds(p * NPIX + base, CH)],
                                 y0b.at[pl.ds(s, CH)], sem_aux)
                pltpu.async_copy(wf.at[pl.ds((p * NPIX + base) * 4, CH * 4)],
                                 wb.at[pl.ds(s * 4, CH * 4)], sem_aux)

        lax.fori_loop(0, P, pb_, None)

    def fire_gathers(ci, q):
        qa = lax.rem(ci, 3)

        def pb_(p, _):
            @pl.when(bit_of(ci, p) != 0)
            def _():
                s = (qa * P + p) * CH
                pltpu.make_async_copy(x0f.at[pl.ds(0, CH)],
                                      x0b.at[pl.ds(s, CH)], sem_aux).wait()
                pltpu.make_async_copy(y0f.at[pl.ds(0, CH)],
                                      y0b.at[pl.ds(s, CH)], sem_aux).wait()
                pltpu.make_async_copy(wf.at[pl.ds(0, CH * 4)],
                                      wb.at[pl.ds(s * 4, CH * 4)], sem_aux).wait()
                pb = p * (PH * PHP)
                r = (q * P + p) * 2
                for v in range(4):
                    xx0 = x0b[pl.ds(s + 16 * v, 16)]
                    yy0 = y0b[pl.ds(s + 16 * v, 16)]
                    xx1 = jnp.minimum(xx0 + 1, PH - 1)
                    yy1 = jnp.minimum(yy0 + 1, PH - 1)
                    c0 = pb + xx0 * PHP
                    c1 = pb + xx1 * PHP
                    idxb[r, pl.ds(16 * v, 16)] = c0 + yy0
                    idxb[r, pl.ds(CH + 16 * v, 16)] = c0 + yy1
                    idxb[r + 1, pl.ds(16 * v, 16)] = c1 + yy0
                    idxb[r + 1, pl.ds(CH + 16 * v, 16)] = c1 + yy1
                g = (q * P + p) * GSLOT
                pltpu.async_copy(table.at[idxb.at[r]],
                                 G.at[pl.ds(g, 2 * CH)], sem_g)
                pltpu.async_copy(table.at[idxb.at[r + 1]],
                                 G.at[pl.ds(g + 2 * CH, 2 * CH)], sem_g)

        lax.fori_loop(0, P, pb_, None)

    def compute(ci, q):
        chunk = wid + ci * NW
        qa = lax.rem(ci, 3)
        a0 = q * (8 * CH)
        d0 = q * CH
        for c in range(8):
            for v in range(4):
                acc[pl.ds(a0 + c * CH + 16 * v, 16)] = zf
        for v in range(4):
            den[pl.ds(d0 + 16 * v, 16)] = zf

        def pb_(p, _):
            @pl.when(bit_of(ci, p) != 0)
            def _():
                g = (q * P + p) * GSLOT
                pltpu.make_async_copy(table.at[pl.ds(0, 2 * CH)],
                                      G.at[pl.ds(g, 2 * CH)], sem_g).wait()
                pltpu.make_async_copy(table.at[pl.ds(0, 2 * CH)],
                                      G.at[pl.ds(g + 2 * CH, 2 * CH)],
                                      sem_g).wait()
                s = (qa * P + p) * CH
                for v in range(4):
                    pix = lane + (16 * v)
                    tw = []
                    for k in range(4):
                        wk = plsc.load_gather(wb, [(s + pix) * 4 + k])
                        tw.append(jnp.where(wk > THR, wk, 0.0))
                    dn = pl.ds(d0 + 16 * v, 16)
                    den[dn] = den[dn] + tw[0] + tw[1] + tw[2] + tw[3]
                    for c in range(8):
                        cv = jnp.full((16,), c, jnp.int32)
                        o = pl.ds(a0 + c * CH + 16 * v, 16)
                        sacc = acc[o]
                        for k in range(4):
                            val = plsc.load_gather(G, [g + k * CH + pix, cv])
                            sacc = sacc + tw[k] * val
                        acc[o] = sacc

        lax.fori_loop(0, P, pb_, None)

        for v in range(4):
            dv = jnp.maximum(den[pl.ds(d0 + 16 * v, 16)], EPS)
            for c in range(8):
                o = pl.ds(a0 + c * CH + 16 * v, 16)
                acc[o] = acc[o] / dv
        pltpu.async_copy(acc.at[pl.ds(a0, 8 * CH)],
                         out.at[pl.ds(chunk * (8 * CH), 8 * CH)], sem_out)

    fire_aux(0)
    fire_gathers(0, 0)

    def main_body(i, _):
        @pl.when(i < CPW)
        def _():
            @pl.when(i > 0)
            def _():
                fire_gathers(i, lax.rem(i, 2))

            @pl.when(i + 1 < CPW)
            def _():
                fire_aux(i + 1)

        @pl.when(i > 0)
        def _():
            @pl.when(i > 2)
            def _():
                pltpu.make_async_copy(acc.at[pl.ds(0, 8 * CH)],
                                      out.at[pl.ds(0, 8 * CH)], sem_out).wait()

            compute(i - 1, lax.rem(i - 1, 2))

    lax.fori_loop(0, CPW + 1, main_body, None)
    for _ in range(2):
        pltpu.make_async_copy(acc.at[pl.ds(0, 8 * CH)],
                              out.at[pl.ds(0, 8 * CH)], sem_out).wait()


_pers2equi_sc = functools.partial(
    pl.kernel,
    out_type=jax.ShapeDtypeStruct((NCHUNK * 8 * CH,), jnp.float32),
    mesh=plsc.VectorSubcoreMesh(core_axis_name="c", subcore_axis_name="s"),
    compiler_params=pltpu.CompilerParams(needs_layout_passes=False,
                                         use_tc_tiling_on_sc=False),
    scratch_types=[
        pltpu.VMEM((2 * P * GSLOT, 8), jnp.float32),
        pltpu.VMEM((2 * P * 2, 2 * CH), jnp.int32),
        pltpu.VMEM((3 * P * CH,), jnp.int32),
        pltpu.VMEM((3 * P * CH,), jnp.int32),
        pltpu.VMEM((3 * P * CH * 4,), jnp.float32),
        pltpu.VMEM((CPW * BM_W,), jnp.int32),
        pltpu.VMEM((2 * 8 * CH,), jnp.float32),
        pltpu.VMEM((2 * CH,), jnp.float32),
        pltpu.SemaphoreType.DMA,
        pltpu.SemaphoreType.DMA,
        pltpu.SemaphoreType.DMA,
    ],
)(_sc_kernel)


def kernel(x, w_list, mask, x0, y0, x1, y1):
    del x1, y1
    xt = jnp.transpose(x[0], (3, 2, 1, 0))
    xt = jnp.pad(xt, ((0, 0), (0, 0), (0, 1), (0, 0)))
    table = xt.reshape(P * PH * PHP, 8)
    wf = w_list.reshape(P * NPIX * 4)
    x0f = x0.astype(jnp.int32).reshape(P * NPIX)
    y0f = y0.astype(jnp.int32).reshape(P * NPIX)
    bm = mask.reshape(P, NCHUNK, CH).max(-1).astype(jnp.int32)
    bm = jnp.pad(bm.T, ((0, 0), (0, BM_W - P)))
    bm = bm.reshape(CPW, NW, BM_W).transpose(1, 0, 2).reshape(NW * CPW * BM_W)
    outf = _pers2equi_sc(table, wf, x0f, y0f, bm)
    out = outf.reshape(NCHUNK, 8, CH).transpose(1, 0, 2)
    return out.reshape(1, 8, H, W)
